# Initial kernel scaffold; baseline (speedup 1.0000x reference)
import jax
import jax.numpy as jnp
from jax.experimental import pallas as pl


def kernel(state, phi, internal_J):
    raise NotImplementedError("write your pallas kernel here")


# speedup vs baseline: 196.5558x; 196.5558x over previous
import math

import jax
import jax.numpy as jnp
from jax.experimental import pallas as pl

_J_IN = 0.38
_J_OUT = 0.38
_GAMMA_PLUS = 1e-3
_DT = 1.0
_TWO_PI = 2.0 * math.pi
_TWO_PI_J_IN = _TWO_PI * _J_IN
_C = 2.0 * _GAMMA_PLUS * _DT * _J_OUT


def _soma_kernel(state_ref, phi_ref, j_ref, out_ref):
    s = jnp.sin(_TWO_PI_J_IN * state_ref[...])
    w = jnp.sin(_TWO_PI * j_ref[...])
    acc = jax.lax.dot_general(
        s,
        w,
        dimension_numbers=(((1,), (1,)), ((), ())),
        preferred_element_type=jnp.float32,
        precision=jax.lax.Precision.HIGHEST,
    )
    out_ref[...] = phi_ref[...] + _C * acc


def kernel(state, phi, internal_J):
    return pl.pallas_call(
        _soma_kernel,
        out_shape=jax.ShapeDtypeStruct(state.shape, state.dtype),
    )(state, phi, internal_J)
